# Initial kernel scaffold; baseline (speedup 1.0000x reference)
import jax
import jax.numpy as jnp
from jax.experimental import pallas as pl


def kernel(author_embedding, paper_embedding, edge_index, edge_weight):
    raise NotImplementedError("write your pallas kernel here")


# speedup vs baseline: 3.2747x; 3.2747x over previous
import functools

import jax
import jax.numpy as jnp
from jax import lax
from jax.experimental import pallas as pl
from jax.experimental.pallas import tpu as pltpu
from jax.experimental.pallas import tpu_sc as plsc

N_A = 5000
N_P = 5000
N = N_A + N_P
D = 128
NUM_LAYERS = 3
NPAD = 10240
E = 320000
CHUNK = 128
NTILES = 32
EPT = 10112
EPAD = EPT * NTILES
ROWS_PER_TILE = NPAD // 16


def _sc_spmm(x, src, dst, w, zeros):
    mesh = plsc.VectorSubcoreMesh(core_axis_name="c", subcore_axis_name="s")

    @functools.partial(
        pl.kernel,
        out_type=jax.ShapeDtypeStruct((2, NPAD, D), jnp.float32),
        mesh=mesh,
        scratch_types=[
            pltpu.VMEM((CHUNK,), jnp.int32),
            pltpu.VMEM((CHUNK,), jnp.int32),
            pltpu.VMEM((CHUNK,), jnp.float32),
            pltpu.VMEM((CHUNK, D), jnp.float32),
            pltpu.VMEM_SHARED((NPAD, D), jnp.float32),
            pltpu.SemaphoreType.DMA,
        ],
    )
    def k(x_hbm, src_hbm, dst_hbm, w_hbm, z_hbm, out_hbm,
          si, di, wv, rows, acc, sem):
        c = lax.axis_index("c")
        s = lax.axis_index("s")
        wid = c * 16 + s
        r0 = s * ROWS_PER_TILE
        pltpu.sync_copy(z_hbm.at[pl.ds(r0, ROWS_PER_TILE)],
                        acc.at[pl.ds(r0, ROWS_PER_TILE)])
        plsc.subcore_barrier()

        def body(g, carry):
            base = wid * EPT + g * CHUNK
            pltpu.sync_copy(src_hbm.at[pl.ds(base, CHUNK)], si)
            pltpu.sync_copy(dst_hbm.at[pl.ds(base, CHUNK)], di)
            pltpu.sync_copy(w_hbm.at[pl.ds(base, CHUNK)], wv)
            pltpu.async_copy(x_hbm.at[si], rows, sem).wait()

            def scale(e16, carry2):
                wvec = wv[pl.ds(e16 * 16, 16)]
                for lane in range(16):
                    wl = wvec[lane]
                    e = e16 * 16 + lane
                    for j in range(D // 16):
                        rows[e, pl.ds(j * 16, 16)] = (
                            rows[e, pl.ds(j * 16, 16)] * wl)
                return carry2

            lax.fori_loop(0, CHUNK // 16, scale, 0)
            pltpu.sync_copy(rows, acc.at[di], add=True)
            return carry

        lax.fori_loop(0, EPT // CHUNK, body, 0)
        plsc.subcore_barrier()
        pltpu.sync_copy(acc.at[pl.ds(r0, ROWS_PER_TILE)],
                        out_hbm.at[c, pl.ds(r0, ROWS_PER_TILE)])

    return k(x, src, dst, w, zeros)


def _tc_combine(p0, p1, s_prev, final):
    BM = 1024
    grid = (NPAD // BM,)
    spec = pl.BlockSpec((BM, D), lambda i: (i, 0))

    if final:
        def body(p0_ref, p1_ref, sp_ref, o_ref):
            o_ref[...] = (sp_ref[...] + p0_ref[...] + p1_ref[...]) * 0.25

        return pl.pallas_call(
            body,
            grid=grid,
            in_specs=[spec, spec, spec],
            out_specs=spec,
            out_shape=jax.ShapeDtypeStruct((NPAD, D), jnp.float32),
        )(p0, p1, s_prev)

    def body(p0_ref, p1_ref, sp_ref, x_ref, s_ref):
        xx = p0_ref[...] + p1_ref[...]
        x_ref[...] = xx
        s_ref[...] = sp_ref[...] + xx

    return pl.pallas_call(
        body,
        grid=grid,
        in_specs=[spec, spec, spec],
        out_specs=[spec, spec],
        out_shape=[jax.ShapeDtypeStruct((NPAD, D), jnp.float32),
                   jax.ShapeDtypeStruct((NPAD, D), jnp.float32)],
    )(p0, p1, s_prev)


def kernel(author_embedding, paper_embedding, edge_index, edge_weight):
    ego = jnp.concatenate([author_embedding, paper_embedding], axis=0)
    pad_e = EPAD - E
    src = jnp.concatenate(
        [edge_index[0].astype(jnp.int32), jnp.zeros((pad_e,), jnp.int32)])
    dst = jnp.concatenate(
        [edge_index[1].astype(jnp.int32), jnp.zeros((pad_e,), jnp.int32)])
    w = jnp.concatenate(
        [edge_weight.astype(jnp.float32), jnp.zeros((pad_e,), jnp.float32)])
    zeros = jnp.zeros((NPAD, D), jnp.float32)

    x = jnp.pad(ego, ((0, NPAD - N), (0, 0)))
    s_run = x
    mean = None
    for layer in range(NUM_LAYERS):
        p = _sc_spmm(x, src, dst, w, zeros)
        if layer == NUM_LAYERS - 1:
            mean = _tc_combine(p[0], p[1], s_run, final=True)
        else:
            x, s_run = _tc_combine(p[0], p[1], s_run, final=False)

    out = mean[:N]
    return (ego, out[:N_A], out[N_A:])
